# baseline (speedup 1.0000x reference)
import functools

import jax
import jax.numpy as jnp
from jax import lax
from jax.experimental import pallas as pl
from jax.experimental.pallas import tpu as pltpu
from jax.experimental.pallas import tpu_sc as plsc

VOCAB = 64001
DIM = 128
HALF = DIM // 2
MAX_LEN = 200
B = 4096
TOK = B * MAX_LEN
EPS = 1e-12


def _transform_body(w_ref, w2_ref, o_ref):
    t = lax.dot_general(
        w_ref[...], w2_ref[...], (((1,), (1,)), ((), ())),
        preferred_element_type=jnp.float32)
    tb = t.astype(jnp.bfloat16)
    a = lax.bitcast_convert_type(tb[:, :HALF], jnp.uint16).astype(jnp.uint32)
    b = lax.bitcast_convert_type(tb[:, HALF:], jnp.uint16).astype(jnp.uint32)
    o_ref[...] = lax.bitcast_convert_type((b << 16) | a, jnp.int32)


def _transform_table(word_table, W2):
    R = 512
    return pl.pallas_call(
        _transform_body,
        grid=(pl.cdiv(VOCAB, R),),
        in_specs=[pl.BlockSpec((R, DIM), lambda i: (i, 0)),
                  pl.BlockSpec((DIM, DIM), lambda i: (0, 0))],
        out_specs=pl.BlockSpec((R, HALF), lambda i: (i, 0)),
        out_shape=jax.ShapeDtypeStruct((VOCAB, HALF), jnp.int32),
    )(word_table, W2)


_NW = 32
_B_PER_W = TOK // _NW
_CH = 512
_NCH = _B_PER_W // _CH


def _sc_gather(table, ids2d):
    mesh = plsc.VectorSubcoreMesh(core_axis_name="c", subcore_axis_name="s")

    @functools.partial(
        pl.kernel,
        out_type=jax.ShapeDtypeStruct((TOK, HALF), jnp.int32),
        mesh=mesh,
        scratch_types=[
            pltpu.VMEM((4, 128), jnp.int32),
            pltpu.VMEM((_CH, HALF), jnp.int32),
            pltpu.SemaphoreType.DMA,
        ],
        compiler_params=pltpu.CompilerParams(use_tc_tiling_on_sc=False),
    )
    def k(t_hbm, ids_hbm, out_hbm, idx_v, rows_v, sem):
        wid = lax.axis_index("s") * 2 + lax.axis_index("c")

        def body(g, carry):
            base = wid * _B_PER_W + g * _CH
            irow = wid * (_B_PER_W // 128) + g * (_CH // 128)
            pltpu.sync_copy(ids_hbm.at[pl.ds(irow, _CH // 128)], idx_v)
            cps = [
                pltpu.async_copy(t_hbm.at[idx_v.at[j]],
                                 rows_v.at[pl.ds(j * 128, 128)], sem)
                for j in range(_CH // 128)
            ]
            for c in cps:
                c.wait()
            pltpu.sync_copy(rows_v, out_hbm.at[pl.ds(base, _CH)])
            return carry

        lax.fori_loop(0, _NCH, body, 0)

    return k(table, ids2d)


def _ln_body(g_ref, seg_ref, pos_ref, typ_ref, gam_ref, bet_ref, o_ref):
    g = g_ref[...]
    lo = lax.bitcast_convert_type(g << 16, jnp.float32)
    hi = lax.bitcast_convert_type(g & jnp.int32(-65536), jnp.float32)
    pos = pos_ref[...]
    seg = seg_ref[...]
    t = typ_ref[...]
    typ_lo = jnp.where(seg == 1, t[1][None, None, :HALF],
                       jnp.where(seg == 2, t[2][None, None, :HALF],
                                 t[0][None, None, :HALF]))
    typ_hi = jnp.where(seg == 1, t[1][None, None, HALF:],
                       jnp.where(seg == 2, t[2][None, None, HALF:],
                                 t[0][None, None, HALF:]))
    xlo = lo + pos[None, :, :HALF] + typ_lo
    xhi = hi + pos[None, :, HALF:] + typ_hi
    mean = (jnp.sum(xlo, axis=-1, keepdims=True)
            + jnp.sum(xhi, axis=-1, keepdims=True)) * (1.0 / DIM)
    dlo = xlo - mean
    dhi = xhi - mean
    var = (jnp.sum(dlo * dlo, axis=-1, keepdims=True)
           + jnp.sum(dhi * dhi, axis=-1, keepdims=True)) * (1.0 / DIM)
    r = lax.rsqrt(var + EPS)
    gam = gam_ref[...]
    bet = bet_ref[...]
    ylo = dlo * r * gam[0][None, None, :HALF] + bet[0][None, None, :HALF]
    yhi = dhi * r * gam[0][None, None, HALF:] + bet[0][None, None, HALF:]
    o_ref[...] = jnp.concatenate([ylo, yhi], axis=-1)


def _ln(gathered, segment_ids, pos_table, type_table, gamma, beta):
    BR = 16
    return pl.pallas_call(
        _ln_body,
        grid=(B // BR,),
        in_specs=[
            pl.BlockSpec((BR, MAX_LEN, HALF), lambda i: (i, 0, 0)),
            pl.BlockSpec((BR, MAX_LEN, 1), lambda i: (i, 0, 0)),
            pl.BlockSpec((MAX_LEN, DIM), lambda i: (0, 0)),
            pl.BlockSpec((3, DIM), lambda i: (0, 0)),
            pl.BlockSpec((1, DIM), lambda i: (0, 0)),
            pl.BlockSpec((1, DIM), lambda i: (0, 0)),
        ],
        out_specs=pl.BlockSpec((BR, MAX_LEN, DIM), lambda i: (i, 0, 0)),
        out_shape=jax.ShapeDtypeStruct((B, MAX_LEN, DIM), jnp.float32),
    )(gathered, segment_ids.reshape(B, MAX_LEN, 1), pos_table, type_table,
      gamma.reshape(1, DIM), beta.reshape(1, DIM))


def kernel(input_ids, segment_ids, word_table, W2, pos_table, type_table,
           gamma, beta):
    table = _transform_table(word_table, W2)
    ids2d = input_ids.astype(jnp.int32).reshape(TOK // 128, 128)
    gathered = _sc_gather(table, ids2d)
    return _ln(gathered.reshape(B, MAX_LEN, HALF), segment_ids.astype(jnp.int32),
               pos_table, type_table, gamma, beta)
